# baseline (speedup 1.0000x reference)
import functools

import jax
import jax.numpy as jnp
from jax import lax
from jax.experimental import pallas as pl
from jax.experimental.pallas import tpu as pltpu
from jax.experimental.pallas import tpu_sc as plsc

NC = 2
NS = 16
NW = NC * NS


@functools.partial(jax.jit, static_argnames=("n_chunks", "chunk"))
def _sc_gather(idx, table, n_chunks, chunk):
    V, D = table.shape
    n = NW * n_chunks * chunk
    mesh = plsc.VectorSubcoreMesh(
        core_axis_name="c", subcore_axis_name="s", num_cores=NC, num_subcores=NS
    )

    n_pairs = n_chunks // 2
    assert n_pairs * 2 == n_chunks

    @functools.partial(
        pl.kernel,
        out_type=jax.ShapeDtypeStruct((n, D), jnp.float32),
        mesh=mesh,
        scratch_types=[
            pltpu.VMEM((n_chunks, chunk), jnp.int32),
            pltpu.VMEM((chunk, D), jnp.float32),
            pltpu.VMEM((chunk, D), jnp.float32),
            pltpu.SemaphoreType.DMA,
            pltpu.SemaphoreType.DMA,
            pltpu.SemaphoreType.DMA,
            pltpu.SemaphoreType.DMA,
        ],
        compiler_params=pltpu.CompilerParams(use_tc_tiling_on_sc=False),
    )
    def k(idx_hbm, tbl_hbm, out_hbm, idx_v, rows0, rows1, g0, g1, o0, o1):
        wid = lax.axis_index("s") * NC + lax.axis_index("c")
        base = wid * (n_chunks * chunk)
        pltpu.sync_copy(idx_hbm.at[wid], idx_v)

        pltpu.async_copy(tbl_hbm.at[idx_v.at[0]], rows0, g0)
        pltpu.async_copy(tbl_hbm.at[idx_v.at[1]], rows1, g1)

        def pair(p, carry):
            j0 = p * 2
            pltpu.make_async_copy(tbl_hbm.at[idx_v.at[j0]], rows0, g0).wait()
            w0 = pltpu.async_copy(
                rows0, out_hbm.at[pl.ds(base + j0 * chunk, chunk)], o0
            )
            pltpu.make_async_copy(tbl_hbm.at[idx_v.at[j0 + 1]], rows1, g1).wait()
            w1 = pltpu.async_copy(
                rows1, out_hbm.at[pl.ds(base + (j0 + 1) * chunk, chunk)], o1
            )
            jn = lax.min(p + 1, n_pairs - 1) * 2
            w0.wait()
            pltpu.async_copy(tbl_hbm.at[idx_v.at[jn]], rows0, g0)
            w1.wait()
            pltpu.async_copy(tbl_hbm.at[idx_v.at[jn + 1]], rows1, g1)
            return carry

        lax.fori_loop(0, n_pairs, pair, 0)
        pltpu.make_async_copy(tbl_hbm.at[idx_v.at[0]], rows0, g0).wait()
        pltpu.make_async_copy(tbl_hbm.at[idx_v.at[1]], rows1, g1).wait()

    return k(idx, table)


def kernel(index, table):
    B, T = index.shape
    V, D = table.shape
    n = B * T
    chunk = 40
    n_chunks = n // (NW * chunk)
    assert NW * n_chunks * chunk == n
    idx = index.reshape(NW, n_chunks, chunk).astype(jnp.int32)
    out = _sc_gather(idx, table, n_chunks, chunk)
    return out.reshape(B, T, D)
